# Initial kernel scaffold; baseline (speedup 1.0000x reference)
import jax
import jax.numpy as jnp
from jax.experimental import pallas as pl


def kernel(x, edge_index, delta_agg, W_mean, b_mean, W_ego, b_ego, W_nb, b_nb, gate_weight, gate_bias):
    raise NotImplementedError("write your pallas kernel here")


# speedup vs baseline: 3.6675x; 3.6675x over previous
import functools
import math

import jax
import jax.numpy as jnp
from jax import lax
from jax.experimental import pallas as pl
from jax.experimental.pallas import tpu as pltpu
from jax.experimental.pallas import tpu_sc as plsc

NC = 2
NS = 16
NW = NC * NS
CH = 128


def _sc_aggregate(x, srcr, dstr, n, n_acc, k_ch):
    d = x.shape[1]
    nz = n_acc // NS

    mesh = plsc.VectorSubcoreMesh(core_axis_name="c", subcore_axis_name="s")

    @functools.partial(
        pl.kernel,
        out_type=(
            jax.ShapeDtypeStruct((NC, n_acc, d), jnp.float32),
            jax.ShapeDtypeStruct((NC * n_acc,), jnp.float32),
        ),
        mesh=mesh,
        scratch_types=[
            pltpu.VMEM((k_ch, CH), jnp.int32),
            pltpu.VMEM((k_ch, CH), jnp.int32),
            pltpu.VMEM((CH, d), jnp.float32),
            pltpu.VMEM((CH,), jnp.float32),
            pltpu.VMEM((n_acc // NS,), jnp.float32),
            pltpu.VMEM_SHARED((n_acc, d), jnp.float32),
            pltpu.VMEM_SHARED((n_acc,), jnp.float32),
            pltpu.SemaphoreType.DMA,
        ],
    )
    def sc_agg(x_hbm, src_hbm, dst_hbm, ns_out, deg_out,
               srcv, dstv, rows, onesb, zb, acc, deg_sh, sem):
        c = lax.axis_index("c")
        s = lax.axis_index("s")
        wid = s * NC + c

        pltpu.sync_copy(src_hbm.at[pl.ds(wid * k_ch, k_ch)], srcv)
        pltpu.sync_copy(dst_hbm.at[pl.ds(wid * k_ch, k_ch)], dstv)

        def zrow(i, carry):
            for cc in range(d // 16):
                rows[i, pl.ds(cc * 16, 16)] = jnp.zeros((16,), jnp.float32)
            return carry
        lax.fori_loop(0, CH, zrow, 0)

        zbase = s * nz
        for kk in range(nz // CH):
            pltpu.sync_copy(rows, acc.at[pl.ds(zbase + kk * CH, CH)])
        zrem = nz % CH
        if zrem:
            pltpu.sync_copy(rows.at[pl.ds(0, zrem)],
                            acc.at[pl.ds(zbase + (nz // CH) * CH, zrem)])

        for cc in range(CH // 16):
            onesb[pl.ds(cc * 16, 16)] = jnp.ones((16,), jnp.float32)
        def zdeg(i, carry):
            zb[pl.ds(i * 16, 16)] = jnp.zeros((16,), jnp.float32)
            return carry
        lax.fori_loop(0, nz // 16, zdeg, 0)
        pltpu.sync_copy(zb, deg_sh.at[pl.ds(s * nz, nz)])

        plsc.subcore_barrier()

        def chunk(j, carry):
            pltpu.async_copy(x_hbm.at[dstv.at[j]], rows, sem).wait()
            pltpu.sync_copy(rows, acc.at[srcv.at[j]], add=True)
            pltpu.sync_copy(onesb, deg_sh.at[srcv.at[j]], add=True)
            return carry
        lax.fori_loop(0, k_ch, chunk, 0)

        plsc.subcore_barrier()

        for kk in range(nz // CH):
            pltpu.sync_copy(acc.at[pl.ds(zbase + kk * CH, CH)],
                            ns_out.at[c, pl.ds(zbase + kk * CH, CH)])
        if zrem:
            ob = zbase + (nz // CH) * CH
            pltpu.sync_copy(acc.at[pl.ds(ob, zrem)],
                            ns_out.at[c, pl.ds(ob, zrem)])

        pltpu.sync_copy(deg_sh.at[pl.ds(s * nz, nz)],
                        deg_out.at[pl.ds(c * n_acc + s * nz, nz)])

    return sc_agg(x, srcr, dstr)


def _tc_body(x_ref, ns_ref, deg_ref, dlt_ref, w_ref, bm_ref, bc_ref, gp_ref,
             out_ref):
    d = x_ref.shape[1]
    ns = ns_ref[0] + ns_ref[1]
    deg = jnp.clip(jnp.sum(deg_ref[...], axis=1), 1.0, None)
    mn = ns * (1.0 / deg)[:, None]
    xm = jnp.concatenate([x_ref[...], mn], axis=1)
    z = jnp.dot(xm, w_ref[...], preferred_element_type=jnp.float32)
    g = jax.nn.sigmoid(gp_ref[0] * dlt_ref[...][:, 0] + gp_ref[1])[:, None]
    h_mean = 0.5 * z[:, :d] + bm_ref[...]
    h_cat = z[:, d:] + bc_ref[...]
    out_ref[...] = h_mean + g * (h_cat - h_mean)


def kernel(x, edge_index, delta_agg, W_mean, b_mean, W_ego, b_ego, W_nb, b_nb,
           gate_weight, gate_bias):
    n, d = x.shape
    e = edge_index.shape[1]

    k_ch = 8 * (-(-e // (NW * CH * 8)))
    e_pad = NW * k_ch * CH
    n_acc = (NS * 16) * (-(-(n + 1) // (NS * 16)))

    src = edge_index[0]
    dst = edge_index[1]
    pad = e_pad - e
    if pad:
        src = jnp.concatenate([src, jnp.full((pad,), n, jnp.int32)])
        dst = jnp.concatenate([dst, jnp.zeros((pad,), jnp.int32)])
    srcr = src.reshape(NW * k_ch, CH)
    dstr = dst.reshape(NW * k_ch, CH)

    ns_p, deg_flat = _sc_aggregate(x, srcr, dstr, n, n_acc, k_ch)
    deg_p = deg_flat.reshape(NC, n_acc).T

    top = jnp.concatenate(
        [W_mean.T, W_ego.T, jnp.zeros((d, d - W_ego.shape[0]), jnp.float32)],
        axis=1)
    bot = jnp.concatenate(
        [W_mean.T, jnp.zeros((d, W_ego.shape[0]), jnp.float32), W_nb.T],
        axis=1)
    wbig = jnp.concatenate([top, bot], axis=0)
    bm = b_mean[None, :]
    bc = jnp.concatenate([b_ego, b_nb])[None, :]
    gp = jnp.stack([gate_weight.astype(jnp.float32),
                    gate_bias.astype(jnp.float32)])
    dlt = delta_agg[:, None]

    r = 1000
    grid = (n // r,)
    h = pl.pallas_call(
        _tc_body,
        grid=grid,
        in_specs=[
            pl.BlockSpec((r, d), lambda i: (i, 0)),
            pl.BlockSpec((NC, r, d), lambda i: (0, i, 0)),
            pl.BlockSpec((r, NC), lambda i: (i, 0)),
            pl.BlockSpec((r, 1), lambda i: (i, 0)),
            pl.BlockSpec((2 * d, 2 * d), lambda i: (0, 0)),
            pl.BlockSpec((1, d), lambda i: (0, 0)),
            pl.BlockSpec((1, d), lambda i: (0, 0)),
            pl.BlockSpec(memory_space=pltpu.SMEM),
        ],
        out_specs=pl.BlockSpec((r, d), lambda i: (i, 0)),
        out_shape=jax.ShapeDtypeStruct((n, d), jnp.float32),
    )(x, ns_p, deg_p, dlt, wbig, bm, bc, gp)
    return h


# speedup vs baseline: 5.6348x; 1.5364x over previous
import functools
import math

import jax
import jax.numpy as jnp
from jax import lax
from jax.experimental import pallas as pl
from jax.experimental.pallas import tpu as pltpu
from jax.experimental.pallas import tpu_sc as plsc

NC = 2
NS = 16
NW = NC * NS
CH = 128
NB = 4
IB = 32


def _sc_aggregate(xs, srcr, dstr, n, n_acc, k_ch):
    hd = xs.shape[2]
    nz = n_acc // NS

    mesh = plsc.VectorSubcoreMesh(core_axis_name="c", subcore_axis_name="s")

    @functools.partial(
        pl.kernel,
        out_type=(
            jax.ShapeDtypeStruct((NC, n_acc, hd), jnp.float32),
            jax.ShapeDtypeStruct((n_acc,), jnp.float32),
        ),
        mesh=mesh,
        scratch_types=[
            pltpu.VMEM((IB, CH), jnp.int32),
            pltpu.VMEM((IB, CH), jnp.int32),
            pltpu.VMEM((NB, CH, hd), jnp.float32),
            pltpu.VMEM((CH,), jnp.float32),
            pltpu.VMEM((n_acc // NS,), jnp.float32),
            pltpu.VMEM_SHARED((n_acc, hd), jnp.float32),
            pltpu.VMEM_SHARED((n_acc,), jnp.float32),
        ] + [pltpu.SemaphoreType.DMA] * NB,
        compiler_params=pltpu.CompilerParams(use_tc_tiling_on_sc=False),
    )
    def sc_agg(xs_hbm, src_hbm, dst_hbm, ns_out, deg_out,
               srcv, dstv, rows, onesb, zb, acc, deg_sh, *sems):
        c = lax.axis_index("c")
        s = lax.axis_index("s")
        xh = xs_hbm.at[c]

        def zrow(i, carry):
            for cc in range(hd // 16):
                rows[0, i, pl.ds(cc * 16, 16)] = jnp.zeros((16,), jnp.float32)
            return carry
        lax.fori_loop(0, CH, zrow, 0)

        zbase = s * nz
        for kk in range(nz // CH):
            pltpu.sync_copy(rows.at[0], acc.at[pl.ds(zbase + kk * CH, CH)])
        zrem = nz % CH
        if zrem:
            pltpu.sync_copy(rows.at[0, pl.ds(0, zrem)],
                            acc.at[pl.ds(zbase + (nz // CH) * CH, zrem)])

        for cc in range(CH // 16):
            onesb[pl.ds(cc * 16, 16)] = jnp.ones((16,), jnp.float32)
        def zdeg(i, carry):
            zb[pl.ds(i * 16, 16)] = jnp.zeros((16,), jnp.float32)
            return carry
        lax.fori_loop(0, nz // 16, zdeg, 0)
        pltpu.sync_copy(zb, deg_sh.at[pl.ds(s * nz, nz)])

        plsc.subcore_barrier()

        for ib in range(k_ch // IB):
            cbase = s * k_ch + ib * IB
            pltpu.sync_copy(src_hbm.at[pl.ds(cbase, IB)], srcv)
            pltpu.sync_copy(dst_hbm.at[pl.ds(cbase, IB)], dstv)

            for b in range(NB):
                pltpu.async_copy(xh.at[dstv.at[b]], rows.at[b], sems[b])

            def group(j0, carry):
                for b in range(NB):
                    j = j0 + b
                    pltpu.make_async_copy(xh.at[dstv.at[j]], rows.at[b],
                                          sems[b]).wait()
                    pltpu.sync_copy(rows.at[b], acc.at[srcv.at[j]], add=True)

                    @pl.when(c == 0)
                    def _():
                        pltpu.sync_copy(onesb, deg_sh.at[srcv.at[j]],
                                        add=True)

                    @pl.when(j + NB < IB)
                    def _():
                        pltpu.async_copy(xh.at[dstv.at[j + NB]], rows.at[b],
                                         sems[b])
                return carry
            lax.fori_loop(0, IB // NB, lambda g, cr: group(g * NB, cr), 0)

        plsc.subcore_barrier()

        for kk in range(nz // CH):
            pltpu.sync_copy(acc.at[pl.ds(zbase + kk * CH, CH)],
                            ns_out.at[c, pl.ds(zbase + kk * CH, CH)])
        if zrem:
            ob = zbase + (nz // CH) * CH
            pltpu.sync_copy(acc.at[pl.ds(ob, zrem)],
                            ns_out.at[c, pl.ds(ob, zrem)])

        @pl.when(c == 0)
        def _():
            pltpu.sync_copy(deg_sh.at[pl.ds(s * nz, nz)],
                            deg_out.at[pl.ds(s * nz, nz)])

    return sc_agg(xs, srcr, dstr)


def _tc_body(x_ref, ns_ref, deg_ref, dlt_ref, w_ref, bm_ref, bc_ref, gp_ref,
             out_ref):
    d = x_ref.shape[1]
    ns = jnp.concatenate([ns_ref[0], ns_ref[1]], axis=1)
    deg = jnp.clip(deg_ref[...][:, 0], 1.0, None)
    mn = ns * (1.0 / deg)[:, None]
    xm = jnp.concatenate([x_ref[...], mn], axis=1)
    z = jnp.dot(xm, w_ref[...], preferred_element_type=jnp.float32)
    g = jax.nn.sigmoid(gp_ref[0] * dlt_ref[...][:, 0] + gp_ref[1])[:, None]
    h_mean = 0.5 * z[:, :d] + bm_ref[...]
    h_cat = z[:, d:] + bc_ref[...]
    out_ref[...] = h_mean + g * (h_cat - h_mean)


def kernel(x, edge_index, delta_agg, W_mean, b_mean, W_ego, b_ego, W_nb, b_nb,
           gate_weight, gate_bias):
    n, d = x.shape
    e = edge_index.shape[1]

    k_ch = IB * (-(-e // (NS * CH * IB)))
    e_pad = NS * k_ch * CH
    n_acc = (NS * 16) * (-(-(n + 1) // (NS * 16)))

    src = edge_index[0]
    dst = edge_index[1]
    pad = e_pad - e
    if pad:
        src = jnp.concatenate([src, jnp.full((pad,), n, jnp.int32)])
        dst = jnp.concatenate([dst, jnp.zeros((pad,), jnp.int32)])
    srcr = src.reshape(NS * k_ch, CH)
    dstr = dst.reshape(NS * k_ch, CH)
    hd = d // NC
    xs = jnp.stack([x[:, c * hd:(c + 1) * hd] for c in range(NC)])

    ns_p, deg_flat = _sc_aggregate(xs, srcr, dstr, n, n_acc, k_ch)
    deg_p = deg_flat[:, None]

    top = jnp.concatenate(
        [W_mean.T, W_ego.T, jnp.zeros((d, d - W_ego.shape[0]), jnp.float32)],
        axis=1)
    bot = jnp.concatenate(
        [W_mean.T, jnp.zeros((d, W_ego.shape[0]), jnp.float32), W_nb.T],
        axis=1)
    wbig = jnp.concatenate([top, bot], axis=0)
    bm = b_mean[None, :]
    bc = jnp.concatenate([b_ego, b_nb])[None, :]
    gp = jnp.stack([gate_weight.astype(jnp.float32),
                    gate_bias.astype(jnp.float32)])
    dlt = delta_agg[:, None]

    r = 1000
    grid = (n // r,)
    h = pl.pallas_call(
        _tc_body,
        grid=grid,
        in_specs=[
            pl.BlockSpec((r, d), lambda i: (i, 0)),
            pl.BlockSpec((NC, r, d // NC), lambda i: (0, i, 0)),
            pl.BlockSpec((r, 1), lambda i: (i, 0)),
            pl.BlockSpec((r, 1), lambda i: (i, 0)),
            pl.BlockSpec((2 * d, 2 * d), lambda i: (0, 0)),
            pl.BlockSpec((1, d), lambda i: (0, 0)),
            pl.BlockSpec((1, d), lambda i: (0, 0)),
            pl.BlockSpec(memory_space=pltpu.SMEM),
        ],
        out_specs=pl.BlockSpec((r, d), lambda i: (i, 0)),
        out_shape=jax.ShapeDtypeStruct((n, d), jnp.float32),
    )(x, ns_p, deg_p, dlt, wbig, bm, bc, gp)
    return h
